# baseline (speedup 1.0000x reference)
import functools

import jax
import jax.numpy as jnp
from jax import lax
from jax.experimental import pallas as pl
from jax.experimental.pallas import tpu as pltpu
from jax.experimental.pallas import tpu_sc as plsc

B = 16384
SEQ = 20
D = 32
NC = 2
NS = 16
NW = NC * NS
BPW = B // NW
CB = 128
NCH = BPW // CB
TPC = CB * SEQ
NSEG = TPC // 128
ACC_ROWS = CB + 1


def _body(tok_hbm, tit_hbm, movie_hbm, tokt_hbm, out_hbm,
          tok2, dst2, gath, tidx, mrows, rcp, comb, res, zero,
          accum, sem_g, sem_m, sem_s):
  cid = lax.axis_index("c")
  sid = lax.axis_index("s")
  wid = sid * NC + cid
  iota = lax.iota(jnp.int32, 16)
  z16 = jnp.zeros((16,), jnp.float32)

  def zloop(i, _):
    zero[i, pl.ds(0, 16)] = z16
    zero[i, pl.ds(16, 16)] = z16
    return _
  lax.fori_loop(jnp.int32(0), jnp.int32(ACC_ROWS), zloop, None)

  acc_base = sid * ACC_ROWS

  def chunk(ch, _):
    gc = wid * NCH + ch

    pltpu.sync_copy(tit_hbm.at[gc], tidx)
    mcp = pltpu.async_copy(movie_hbm.at[tidx], mrows, sem_m)
    pltpu.sync_copy(tok_hbm.at[pl.ds(gc * NSEG, NSEG)], tok2)

    gcps = [
        pltpu.async_copy(tokt_hbm.at[tok2.at[jnp.int32(j)]],
                         gath.at[pl.ds(j * 128, 128)], sem_g)
        for j in range(NSEG)
    ]

    def dstloop(g, _):
      j = lax.div(g, jnp.int32(8))
      l = g - j * 8
      tok = tok2[j, pl.ds(l * 16, 16)]
      flat = g * 16 + iota
      row = lax.div(flat, jnp.full((16,), SEQ, jnp.int32))
      dst = jnp.where(tok != 0, row, jnp.int32(CB)) + acc_base
      dst2[j, pl.ds(l * 16, 16)] = dst
      return _
    lax.fori_loop(jnp.int32(0), jnp.int32(TPC // 16), dstloop, None)

    def cloop(g, _):
      cnt = jnp.zeros((16,), jnp.int32)
      base_flat = (g * 16 + iota) * SEQ
      for t in range(SEQ):
        flat = base_flat + t
        jj = lax.shift_right_logical(flat, jnp.full((16,), 7, jnp.int32))
        cc = flat - jj * 128
        v = plsc.load_gather(tok2, [jj, cc])
        cnt = cnt + (v != 0).astype(jnp.int32)
      cntf = jnp.maximum(cnt.astype(jnp.float32), 1.0)
      rcp[pl.ds(g * 16, 16)] = 1.0 / cntf
      return _
    lax.fori_loop(jnp.int32(0), jnp.int32(CB // 16), cloop, None)

    pltpu.sync_copy(zero, accum.at[pl.ds(acc_base, ACC_ROWS)])
    for cp in gcps:
      cp.wait()
    scps = [
        pltpu.async_copy(gath.at[pl.ds(j * 128, 128)],
                         accum.at[dst2.at[jnp.int32(j)]], sem_s, add=True)
        for j in range(NSEG)
    ]
    for cp in scps:
      cp.wait()

    pltpu.sync_copy(accum.at[pl.ds(acc_base, CB)], res)
    mcp.wait()

    def floop(r, _):
      rb = plsc.load_gather(rcp, [jnp.full((16,), r, jnp.int32)])
      for c in range(D // 16):
        comb[r, pl.ds(c * 16, 16)] = mrows[r, pl.ds(c * 16, 16)]
        comb[r, pl.ds(D + c * 16, 16)] = res[r, pl.ds(c * 16, 16)] * rb
      return _
    lax.fori_loop(jnp.int32(0), jnp.int32(CB), floop, None)

    pltpu.sync_copy(comb, out_hbm.at[pl.ds(gc * CB, CB)])
    return _

  lax.fori_loop(jnp.int32(0), jnp.int32(NCH), chunk, None)


@jax.jit
def _run(tok3, tit2, movie_table, token_table):
  mesh = plsc.VectorSubcoreMesh(core_axis_name="c", subcore_axis_name="s",
                                num_cores=NC, num_subcores=NS)
  f = functools.partial(
      pl.kernel,
      out_type=jax.ShapeDtypeStruct((B, 2 * D), jnp.float32),
      mesh=mesh,
      compiler_params=pltpu.CompilerParams(needs_layout_passes=False,
                                           use_tc_tiling_on_sc=False),
      scratch_types=[
          pltpu.VMEM((NSEG, 128), jnp.int32),
          pltpu.VMEM((NSEG, 128), jnp.int32),
          pltpu.VMEM((TPC, D), jnp.float32),
          pltpu.VMEM((CB,), jnp.int32),
          pltpu.VMEM((CB, D), jnp.float32),
          pltpu.VMEM((CB,), jnp.float32),
          pltpu.VMEM((CB, 2 * D), jnp.float32),
          pltpu.VMEM((CB, D), jnp.float32),
          pltpu.VMEM((ACC_ROWS, D), jnp.float32),
          pltpu.VMEM_SHARED((NS * ACC_ROWS, D), jnp.float32),
          pltpu.SemaphoreType.DMA,
          pltpu.SemaphoreType.DMA,
          pltpu.SemaphoreType.DMA,
      ],
  )(_body)
  return f(tok3, tit2, movie_table, token_table)


def kernel(titles, title_tokens, movie_table, token_table):
  tok2d = title_tokens.reshape(B * SEQ // 128, 128).astype(jnp.int32)
  tit2d = titles.reshape(B // CB, CB).astype(jnp.int32)
  return _run(tok2d, tit2d, movie_table.astype(jnp.float32),
              token_table.astype(jnp.float32))


# speedup vs baseline: 1.0997x; 1.0997x over previous
import functools

import jax
import jax.numpy as jnp
from jax import lax
from jax.experimental import pallas as pl
from jax.experimental.pallas import tpu as pltpu
from jax.experimental.pallas import tpu_sc as plsc

B = 16384
SEQ = 20
D = 32
NC = 2
NS = 16
NW = NC * NS
BPW = B // NW
CB = 128
NCH = BPW // CB
TPC = CB * SEQ
NSEG = TPC // 128
ACC_ROWS = CB + 1


def _body(tok_hbm, tit_hbm, movie_hbm, tokt_hbm, out_hbm,
          tok2, dst2, gath, tidx, mrows, rcp, comb, res, zero,
          accum, sem_g, sem_s, sem_m0, sem_m1):
  cid = lax.axis_index("c")
  sid = lax.axis_index("s")
  wid = sid * NC + cid
  iota = lax.iota(jnp.int32, 16)
  z16 = jnp.zeros((16,), jnp.float32)
  sem_m = [sem_m0, sem_m1]

  def zloop(i, _):
    zero[i, pl.ds(0, 16)] = z16
    zero[i, pl.ds(16, 16)] = z16
    return _
  lax.fori_loop(jnp.int32(0), jnp.int32(ACC_ROWS), zloop, None)

  acc_base = sid * ACC_ROWS

  def load_chunk(ch, p):
    gc = wid * NCH + ch
    pltpu.sync_copy(tit_hbm.at[gc], tidx.at[jnp.int32(p)])
    pltpu.sync_copy(tok_hbm.at[pl.ds(gc * NSEG, NSEG)],
                    tok2.at[jnp.int32(p)])

  def fire_gathers(p):
    m = pltpu.async_copy(movie_hbm.at[tidx.at[jnp.int32(p)]],
                         mrows.at[jnp.int32(p)], sem_m[p])
    gs = [
        pltpu.async_copy(tokt_hbm.at[tok2.at[jnp.int32(p), jnp.int32(j)]],
                         gath.at[pl.ds(j * 128, 128)], sem_g)
        for j in range(NSEG)
    ]
    return m, gs

  load_chunk(0, 0)
  mcp, gcps = fire_gathers(0)

  for ch in range(NCH):
    p = ch % 2
    q = (ch + 1) % 2
    gc = wid * NCH + ch

    def dstloop(g, _):
      j = lax.div(g, jnp.int32(8))
      l = g - j * 8
      tok = tok2[p, j, pl.ds(l * 16, 16)]
      flat = g * 16 + iota
      row = lax.div(flat, jnp.full((16,), SEQ, jnp.int32))
      dst = jnp.where(tok != 0, row, jnp.int32(CB)) + acc_base
      dst2[j, pl.ds(l * 16, 16)] = dst
      return _
    lax.fori_loop(jnp.int32(0), jnp.int32(TPC // 16), dstloop, None)

    def cloop(g, _):
      cnt = jnp.zeros((16,), jnp.int32)
      base_flat = (g * 16 + iota) * SEQ
      for t in range(SEQ):
        flat = base_flat + t
        jj = lax.shift_right_logical(flat, jnp.full((16,), 7, jnp.int32))
        cc = flat - jj * 128
        pp = jnp.full((16,), p, jnp.int32)
        v = plsc.load_gather(tok2, [pp, jj, cc])
        cnt = cnt + (v != 0).astype(jnp.int32)
      cntf = jnp.maximum(cnt.astype(jnp.float32), 1.0)
      rcp[p, pl.ds(g * 16, 16)] = 1.0 / cntf
      return _
    lax.fori_loop(jnp.int32(0), jnp.int32(CB // 16), cloop, None)

    pltpu.sync_copy(zero, accum.at[pl.ds(acc_base, ACC_ROWS)])
    scps = []
    for j in range(NSEG):
      gcps[j].wait()
      scps.append(
          pltpu.async_copy(gath.at[pl.ds(j * 128, 128)],
                           accum.at[dst2.at[jnp.int32(j)]], sem_s,
                           add=True))

    if ch + 1 < NCH:
      load_chunk(ch + 1, q)
    for cp in scps:
      cp.wait()
    if ch + 1 < NCH:
      mcp_n, gcps_n = fire_gathers(q)

    pltpu.sync_copy(accum.at[pl.ds(acc_base, CB)], res)
    mcp.wait()

    def floop(r, _):
      pp = jnp.full((16,), p, jnp.int32)
      rb = plsc.load_gather(rcp, [pp, jnp.full((16,), r, jnp.int32)])
      for c in range(D // 16):
        comb[r, pl.ds(c * 16, 16)] = mrows[p, r, pl.ds(c * 16, 16)]
        comb[r, pl.ds(D + c * 16, 16)] = res[r, pl.ds(c * 16, 16)] * rb
      return _
    lax.fori_loop(jnp.int32(0), jnp.int32(CB), floop, None)

    pltpu.sync_copy(comb, out_hbm.at[pl.ds(gc * CB, CB)])
    if ch + 1 < NCH:
      mcp, gcps = mcp_n, gcps_n


@jax.jit
def _run(tok2d, tit2d, movie_table, token_table):
  mesh = plsc.VectorSubcoreMesh(core_axis_name="c", subcore_axis_name="s",
                                num_cores=NC, num_subcores=NS)
  f = functools.partial(
      pl.kernel,
      out_type=jax.ShapeDtypeStruct((B, 2 * D), jnp.float32),
      mesh=mesh,
      compiler_params=pltpu.CompilerParams(needs_layout_passes=False,
                                           use_tc_tiling_on_sc=False),
      scratch_types=[
          pltpu.VMEM((2, NSEG, 128), jnp.int32),
          pltpu.VMEM((NSEG, 128), jnp.int32),
          pltpu.VMEM((TPC, D), jnp.float32),
          pltpu.VMEM((2, CB), jnp.int32),
          pltpu.VMEM((2, CB, D), jnp.float32),
          pltpu.VMEM((2, CB), jnp.float32),
          pltpu.VMEM((CB, 2 * D), jnp.float32),
          pltpu.VMEM((CB, D), jnp.float32),
          pltpu.VMEM((ACC_ROWS, D), jnp.float32),
          pltpu.VMEM_SHARED((NS * ACC_ROWS, D), jnp.float32),
          pltpu.SemaphoreType.DMA,
          pltpu.SemaphoreType.DMA,
          pltpu.SemaphoreType.DMA,
          pltpu.SemaphoreType.DMA,
      ],
  )(_body)
  return f(tok2d, tit2d, movie_table, token_table)


def kernel(titles, title_tokens, movie_table, token_table):
  tok2d = title_tokens.reshape(B * SEQ // 128, 128).astype(jnp.int32)
  tit2d = titles.reshape(B // CB, CB).astype(jnp.int32)
  return _run(tok2d, tit2d, movie_table.astype(jnp.float32),
              token_table.astype(jnp.float32))


# speedup vs baseline: 1.2626x; 1.1482x over previous
import functools

import jax
import jax.numpy as jnp
from jax import lax
from jax.experimental import pallas as pl
from jax.experimental.pallas import tpu as pltpu
from jax.experimental.pallas import tpu_sc as plsc

B = 16384
SEQ = 20
D = 32
NC = 2
NS = 16
NW = NC * NS
BPW = B // NW
CB = 128
NCH = BPW // CB
TPC = CB * SEQ
NSEG = TPC // 128
ACC_ROWS = CB + 1

_MESH = plsc.VectorSubcoreMesh(core_axis_name="c", subcore_axis_name="s",
                               num_cores=NC, num_subcores=NS)
_PARAMS = pltpu.CompilerParams(needs_layout_passes=False,
                               use_tc_tiling_on_sc=False)


def _pool_body(tok_hbm, tokt_hbm, text_hbm,
               tok2, dst2, gath, rcp, comb4, res, zero,
               accum, sem_g, sem_s):
  cid = lax.axis_index("c")
  sid = lax.axis_index("s")
  wid = sid * NC + cid
  iota = lax.iota(jnp.int32, 16)
  z16 = jnp.zeros((16,), jnp.float32)

  def zloop(i, _):
    zero[i, pl.ds(0, 16)] = z16
    zero[i, pl.ds(16, 16)] = z16
    return _
  lax.fori_loop(jnp.int32(0), jnp.int32(ACC_ROWS), zloop, None)

  acc_base = sid * ACC_ROWS

  def load_chunk(ch, p):
    gc = wid * NCH + ch
    pltpu.sync_copy(tok_hbm.at[pl.ds(gc * NSEG, NSEG)],
                    tok2.at[jnp.int32(p)])

  def fire_gathers(p):
    return [
        pltpu.async_copy(tokt_hbm.at[tok2.at[jnp.int32(p), jnp.int32(j)]],
                         gath.at[pl.ds(j * 128, 128)], sem_g)
        for j in range(NSEG)
    ]

  load_chunk(0, 0)
  gcps = fire_gathers(0)

  for ch in range(NCH):
    p = ch % 2
    q = (ch + 1) % 2
    gc = wid * NCH + ch

    def dstloop(g, _):
      j = lax.div(g, jnp.int32(8))
      l = g - j * 8
      tok = tok2[p, j, pl.ds(l * 16, 16)]
      flat = g * 16 + iota
      row = lax.div(flat, jnp.full((16,), SEQ, jnp.int32))
      dst = jnp.where(tok != 0, row, jnp.int32(CB)) + acc_base
      dst2[j, pl.ds(l * 16, 16)] = dst
      return _
    lax.fori_loop(jnp.int32(0), jnp.int32(TPC // 16), dstloop, None)

    def cloop(g, _):
      cnt = jnp.zeros((16,), jnp.int32)
      base_flat = (g * 16 + iota) * SEQ
      for t in range(SEQ):
        flat = base_flat + t
        jj = lax.shift_right_logical(flat, jnp.full((16,), 7, jnp.int32))
        cc = flat - jj * 128
        pp = jnp.full((16,), p, jnp.int32)
        v = plsc.load_gather(tok2, [pp, jj, cc])
        cnt = cnt + (v != 0).astype(jnp.int32)
      cntf = jnp.maximum(cnt.astype(jnp.float32), 1.0)
      rcp[pl.ds(g * 16, 16)] = 1.0 / cntf
      return _
    lax.fori_loop(jnp.int32(0), jnp.int32(CB // 16), cloop, None)

    pltpu.sync_copy(zero, accum.at[pl.ds(acc_base, ACC_ROWS)])
    scps = []
    for j in range(NSEG):
      gcps[j].wait()
      scps.append(
          pltpu.async_copy(gath.at[pl.ds(j * 128, 128)],
                           accum.at[dst2.at[jnp.int32(j)]], sem_s,
                           add=True))

    if ch + 1 < NCH:
      load_chunk(ch + 1, q)
    for cp in scps:
      cp.wait()
    if ch + 1 < NCH:
      gcps_n = fire_gathers(q)

    pltpu.sync_copy(accum.at[pl.ds(acc_base, CB)], res)

    def floop(r, _):
      rb = plsc.load_gather(rcp, [jnp.full((16,), r, jnp.int32)])
      r4 = lax.div(r, jnp.int32(4))
      c4 = (r - r4 * 4) * D
      for c in range(D // 16):
        comb4[r4, pl.ds(c4 + c * 16, 16)] = res[r, pl.ds(c * 16, 16)] * rb
      return _
    lax.fori_loop(jnp.int32(0), jnp.int32(CB), floop, None)

    pltpu.sync_copy(comb4, text_hbm.at[pl.ds(gc * (CB // 4), CB // 4)])
    if ch + 1 < NCH:
      gcps = gcps_n


def _movie_body(tit_hbm, movie_hbm, text_hbm, out_hbm,
                tidx, mrows, text4, comb, sem_m):
  cid = lax.axis_index("c")
  sid = lax.axis_index("s")
  wid = sid * NC + cid
  base = wid * BPW

  pltpu.sync_copy(tit_hbm.at[pl.ds(wid * (BPW // 128), BPW // 128)], tidx)
  mcps = [
      pltpu.async_copy(movie_hbm.at[tidx.at[jnp.int32(j)]],
                       mrows.at[pl.ds(j * 128, 128)], sem_m)
      for j in range(BPW // 128)
  ]
  pltpu.sync_copy(text_hbm.at[pl.ds(wid * (BPW // 4), BPW // 4)], text4)
  for cp in mcps:
    cp.wait()

  def floop(r, _):
    r4 = lax.div(r, jnp.int32(4))
    c4 = (r - r4 * 4) * D
    for c in range(D // 16):
      comb[r, pl.ds(c * 16, 16)] = mrows[r, pl.ds(c * 16, 16)]
      comb[r, pl.ds(D + c * 16, 16)] = text4[r4, pl.ds(c4 + c * 16, 16)]
    return _
  lax.fori_loop(jnp.int32(0), jnp.int32(BPW), floop, None)

  pltpu.sync_copy(comb, out_hbm.at[pl.ds(base, BPW)])


@jax.jit
def _run(tok2d, tit2d, movie_table, token_table):
  pool = functools.partial(
      pl.kernel,
      out_type=jax.ShapeDtypeStruct((B // 4, 128), jnp.float32),
      mesh=_MESH,
      compiler_params=_PARAMS,
      scratch_types=[
          pltpu.VMEM((2, NSEG, 128), jnp.int32),
          pltpu.VMEM((NSEG, 128), jnp.int32),
          pltpu.VMEM((TPC, D), jnp.float32),
          pltpu.VMEM((CB,), jnp.float32),
          pltpu.VMEM((CB // 4, 128), jnp.float32),
          pltpu.VMEM((CB, D), jnp.float32),
          pltpu.VMEM((ACC_ROWS, D), jnp.float32),
          pltpu.VMEM_SHARED((NS * ACC_ROWS, D), jnp.float32),
          pltpu.SemaphoreType.DMA,
          pltpu.SemaphoreType.DMA,
      ],
  )(_pool_body)
  text = pool(tok2d, token_table)

  movie = functools.partial(
      pl.kernel,
      out_type=jax.ShapeDtypeStruct((B, 2 * D), jnp.float32),
      mesh=_MESH,
      compiler_params=_PARAMS,
      scratch_types=[
          pltpu.VMEM((BPW // 128, 128), jnp.int32),
          pltpu.VMEM((BPW, D), jnp.float32),
          pltpu.VMEM((BPW // 4, 128), jnp.float32),
          pltpu.VMEM((BPW, 2 * D), jnp.float32),
          pltpu.SemaphoreType.DMA,
      ],
  )(_movie_body)
  return movie(tit2d, movie_table, text)


def kernel(titles, title_tokens, movie_table, token_table):
  tok2d = title_tokens.reshape(B * SEQ // 128, 128).astype(jnp.int32)
  tit2d = titles.reshape(B // 128, 128).astype(jnp.int32)
  return _run(tok2d, tit2d, movie_table.astype(jnp.float32),
              token_table.astype(jnp.float32))
